# Initial kernel scaffold; baseline (speedup 1.0000x reference)
import jax
import jax.numpy as jnp
from jax.experimental import pallas as pl


def kernel(input, embedding, position_emb):
    raise NotImplementedError("write your pallas kernel here")


# speedup vs baseline: 1.0545x; 1.0545x over previous
import functools

import jax
import jax.numpy as jnp
from jax import lax
from jax.experimental import pallas as pl
from jax.experimental.pallas import tpu as pltpu
from jax.experimental.pallas import tpu_sc as plsc

N_CLASSES = 1024 + 3 * 128 + 1
DIM = 1024
BATCH = 4
SEQ = 2048
TOTAL = BATCH * SEQ

NC = 2
NS = 16
NW = NC * NS
PER_W = TOTAL // NW
CHUNK = 16
NCHUNK = PER_W // CHUNK
LANES = 16


def _sc_body(idx_hbm, table_hbm, pos_hbm, emb_out, mask_out,
             idx_v, mask_v, rbuf, pbuf, gsem):
    wid = lax.axis_index("s") * NC + lax.axis_index("c")
    base = wid * PER_W
    pos_base = lax.rem(base, SEQ)

    pltpu.sync_copy(idx_hbm.at[wid], idx_v)
    for c in range(NCHUNK):
        v = idx_v[c]
        mask_v[c] = jnp.where(v == N_CLASSES - 1, 1, 0).astype(jnp.int32)
    pltpu.sync_copy(mask_v, mask_out.at[wid])

    for c in range(NCHUNK):
        row0 = base + c * CHUNK
        prow0 = pos_base + c * CHUNK
        gather = pltpu.async_copy(table_hbm.at[idx_v.at[c]], rbuf, gsem)
        pltpu.sync_copy(pos_hbm.at[pl.ds(prow0, CHUNK)], pbuf)
        gather.wait()

        def addbody(r, carry):
            for jc in range(DIM // LANES):
                sl = pl.ds(jc * LANES, LANES)
                rbuf[r, sl] = rbuf[r, sl] + pbuf[r, sl]
            return carry

        lax.fori_loop(0, CHUNK, addbody, 0)
        pltpu.sync_copy(rbuf, emb_out.at[pl.ds(row0, CHUNK)])


@jax.jit
def _content_emb(flat_idx, embedding, pos2d):
    mesh = plsc.VectorSubcoreMesh(
        core_axis_name="c", subcore_axis_name="s",
        num_cores=NC, num_subcores=NS)
    run = pl.kernel(
        _sc_body,
        out_type=[
            jax.ShapeDtypeStruct((TOTAL, DIM), jnp.float32),
            jax.ShapeDtypeStruct((NW, NCHUNK, CHUNK), jnp.int32),
        ],
        mesh=mesh,
        scratch_types=[
            pltpu.VMEM((NCHUNK, CHUNK), jnp.int32),
            pltpu.VMEM((NCHUNK, CHUNK), jnp.int32),
            pltpu.VMEM((CHUNK, DIM), jnp.float32),
            pltpu.VMEM((CHUNK, DIM), jnp.float32),
            pltpu.SemaphoreType.DMA,
        ],
    )
    return run(flat_idx.reshape(NW, NCHUNK, CHUNK), embedding, pos2d)


def kernel(input, embedding, position_emb):
    emb_flat, mask_flat = _content_emb(
        input.reshape(TOTAL), embedding, position_emb.reshape(SEQ, DIM))
    return (emb_flat.reshape(BATCH, SEQ, DIM),
            mask_flat.reshape(BATCH, SEQ))


# speedup vs baseline: 1.4418x; 1.3672x over previous
import functools

import jax
import jax.numpy as jnp
from jax import lax
from jax.experimental import pallas as pl
from jax.experimental.pallas import tpu as pltpu
from jax.experimental.pallas import tpu_sc as plsc

N_CLASSES = 1024 + 3 * 128 + 1
DIM = 1024
BATCH = 4
SEQ = 2048
TOTAL = BATCH * SEQ

NC = 2
NS = 16
NW = NC * NS
PER_W = TOTAL // NW
CHUNK = 16
NCHUNK = PER_W // CHUNK
LANES = 16


def _sc_body(idx_hbm, table_hbm, pos_hbm, emb_out, mask_out,
             idx_v, mask_v, rbuf0, rbuf1, pbuf0, pbuf1,
             gsem0, gsem1, psem0, psem1, osem0, osem1):
    wid = lax.axis_index("s") * NC + lax.axis_index("c")
    base = wid * PER_W
    pos_base = lax.rem(base, SEQ)

    rbufs = (rbuf0, rbuf1)
    pbufs = (pbuf0, pbuf1)
    gsems = (gsem0, gsem1)
    psems = (psem0, psem1)
    osems = (osem0, osem1)

    pltpu.sync_copy(idx_hbm.at[wid], idx_v)
    for c in range(NCHUNK):
        v = idx_v[c]
        mask_v[c] = jnp.where(v == N_CLASSES - 1, 1, 0).astype(jnp.int32)
    pltpu.sync_copy(mask_v, mask_out.at[wid])

    ohandles = [None] * NCHUNK
    ghandles = [None] * NCHUNK
    phandles = [None] * NCHUNK

    def issue(c):
        s = c % 2
        if c >= 2:
            ohandles[c - 2].wait()
        ghandles[c] = pltpu.async_copy(
            table_hbm.at[idx_v.at[c]], rbufs[s], gsems[s])
        phandles[c] = pltpu.async_copy(
            pos_hbm.at[pl.ds(pos_base + c * CHUNK, CHUNK)], pbufs[s],
            psems[s])

    issue(0)
    for c in range(NCHUNK):
        if c + 1 < NCHUNK:
            issue(c + 1)
        s = c % 2
        ghandles[c].wait()
        phandles[c].wait()
        rb, pb = rbufs[s], pbufs[s]

        def addbody(r, carry):
            for jc in range(DIM // LANES):
                sl = pl.ds(jc * LANES, LANES)
                rb[r, sl] = rb[r, sl] + pb[r, sl]
            return carry

        lax.fori_loop(0, CHUNK, addbody, 0)
        ohandles[c] = pltpu.async_copy(
            rb, emb_out.at[pl.ds(base + c * CHUNK, CHUNK)], osems[s])
    ohandles[NCHUNK - 2].wait()
    ohandles[NCHUNK - 1].wait()


@jax.jit
def _content_emb(flat_idx, embedding, pos2d):
    mesh = plsc.VectorSubcoreMesh(
        core_axis_name="c", subcore_axis_name="s",
        num_cores=NC, num_subcores=NS)
    run = pl.kernel(
        _sc_body,
        out_type=[
            jax.ShapeDtypeStruct((TOTAL, DIM), jnp.float32),
            jax.ShapeDtypeStruct((NW, NCHUNK, CHUNK), jnp.int32),
        ],
        mesh=mesh,
        scratch_types=[
            pltpu.VMEM((NCHUNK, CHUNK), jnp.int32),
            pltpu.VMEM((NCHUNK, CHUNK), jnp.int32),
            pltpu.VMEM((CHUNK, DIM), jnp.float32),
            pltpu.VMEM((CHUNK, DIM), jnp.float32),
            pltpu.VMEM((CHUNK, DIM), jnp.float32),
            pltpu.VMEM((CHUNK, DIM), jnp.float32),
            pltpu.SemaphoreType.DMA,
            pltpu.SemaphoreType.DMA,
            pltpu.SemaphoreType.DMA,
            pltpu.SemaphoreType.DMA,
            pltpu.SemaphoreType.DMA,
            pltpu.SemaphoreType.DMA,
        ],
    )
    return run(flat_idx.reshape(NW, NCHUNK, CHUNK), embedding, pos2d)


def kernel(input, embedding, position_emb):
    emb_flat, mask_flat = _content_emb(
        input.reshape(TOTAL), embedding, position_emb.reshape(SEQ, DIM))
    return (emb_flat.reshape(BATCH, SEQ, DIM),
            mask_flat.reshape(BATCH, SEQ))
